# baseline (speedup 1.0000x reference)
import functools

import jax
import jax.numpy as jnp
from jax import lax
from jax.experimental import pallas as pl
from jax.experimental.pallas import tpu as pltpu
from jax.experimental.pallas import tpu_sc as plsc

N_NODES = 10000
N_PAD = 10240
D = 128
DH = D // 2
NC, NS = 2, 16
NTILE = NC * NS
CHUNK = 64
NCHUNK_DEG = 160
NCHUNK = 320
E_PAD = NS * NCHUNK * CHUNK
ROWS_PER_TILE = N_PAD // NS
_NBUF = 10
_NBUF_HBM = 0

_mesh = plsc.VectorSubcoreMesh(core_axis_name="c", subcore_axis_name="s")


@functools.partial(
    pl.kernel,
    out_type=jax.ShapeDtypeStruct((NC, N_PAD), jnp.float32),
    mesh=_mesh,
    scratch_types=[
        pltpu.VMEM((NCHUNK_DEG, CHUNK), jnp.int32),
        pltpu.VMEM((CHUNK,), jnp.float32),
        pltpu.VMEM((ROWS_PER_TILE,), jnp.float32),
        pltpu.VMEM_SHARED((N_PAD,), jnp.float32),
    ],
)
def _deg_kernel(dst_hbm, out_hbm, dst_v, ones_v, zeros_v, acc):
    c = lax.axis_index("c")
    s = lax.axis_index("s")
    wid = c * NS + s
    for k in range(CHUNK // 16):
        ones_v[pl.ds(k * 16, 16)] = jnp.ones((16,), jnp.float32)
    for k in range(ROWS_PER_TILE // 16):
        zeros_v[pl.ds(k * 16, 16)] = jnp.zeros((16,), jnp.float32)
    pltpu.sync_copy(zeros_v, acc.at[pl.ds(s * ROWS_PER_TILE, ROWS_PER_TILE)])
    pltpu.sync_copy(dst_hbm.at[wid], dst_v)
    plsc.subcore_barrier()

    def body(j, carry):
        pltpu.sync_copy(ones_v, acc.at[dst_v.at[j]], add=True)
        return carry

    lax.fori_loop(0, NCHUNK_DEG, body, 0)
    plsc.subcore_barrier()
    pltpu.sync_copy(acc.at[pl.ds(s * ROWS_PER_TILE, ROWS_PER_TILE)],
                    out_hbm.at[c, pl.ds(s * ROWS_PER_TILE, ROWS_PER_TILE)])


@functools.partial(
    pl.kernel,
    out_type=[jax.ShapeDtypeStruct((N_PAD, DH), jnp.float32),
              jax.ShapeDtypeStruct((N_PAD, DH), jnp.float32)],
    mesh=_mesh,
    scratch_types=[
        [pltpu.VMEM((2, CHUNK), jnp.int32)] * _NBUF,
        [pltpu.VMEM((CHUNK, DH), jnp.float32)] * _NBUF,
        pltpu.VMEM_SHARED((N_PAD, DH), jnp.float32),
        pltpu.VMEM_SHARED((N_PAD, DH), jnp.float32),
        [pltpu.SemaphoreType.DMA] * _NBUF,
        [pltpu.SemaphoreType.DMA] * _NBUF,
        [pltpu.SemaphoreType.DMA] * _NBUF,
    ],
    compiler_params=pltpu.CompilerParams(use_tc_tiling_on_sc=False),
)
def _agg_kernel(z0_hbm, z1_hbm, idx_hbm, out0_hbm, out1_hbm, idxb, rows, z_sh,
                acc, isem, gsem, ssem):
    c = lax.axis_index("c")
    s = lax.axis_index("s")
    row0 = s * ROWS_PER_TILE
    sl = pl.ds(row0, ROWS_PER_TILE)

    @pl.when(c == 0)
    def _():
        pltpu.sync_copy(z0_hbm.at[sl], z_sh.at[sl])
        pltpu.sync_copy(z0_hbm.at[sl], acc.at[sl])

    @pl.when(c != 0)
    def _():
        pltpu.sync_copy(z1_hbm.at[sl], z_sh.at[sl])
        pltpu.sync_copy(z1_hbm.at[sl], acc.at[sl])

    def idx_dma(j, b):
        return pltpu.async_copy(idx_hbm.at[s, j], idxb[b], isem[b])

    def gather(b):
        if b < _NBUF_HBM:
            @pl.when(c == 0)
            def _():
                pltpu.async_copy(z0_hbm.at[idxb[b].at[0]], rows[b], gsem[b])

            @pl.when(c != 0)
            def _():
                pltpu.async_copy(z1_hbm.at[idxb[b].at[0]], rows[b], gsem[b])

            return pltpu.make_async_copy(z0_hbm.at[idxb[b].at[0]], rows[b],
                                         gsem[b])
        return pltpu.async_copy(z_sh.at[idxb[b].at[0]], rows[b], gsem[b])

    for b in range(_NBUF):
        idx_dma(b, b)
    plsc.subcore_barrier()
    gd = []
    for b in range(_NBUF):
        pltpu.make_async_copy(idx_hbm.at[s, 0], idxb[b], isem[b]).wait()
        gd.append(gather(b))

    def body(i, carry):
        j = _NBUF * i
        sd = []
        for b in range(_NBUF):
            gd[b].wait()
            sd.append(pltpu.async_copy(rows[b], acc.at[idxb[b].at[1]],
                                       ssem[b], add=True))
        for b in range(_NBUF):
            sd[b].wait()
            idx_dma(jnp.minimum(j + _NBUF + b, NCHUNK - _NBUF + b), b)
        for b in range(_NBUF):
            pltpu.make_async_copy(idx_hbm.at[s, 0], idxb[b], isem[b]).wait()
            gather(b)
        return carry

    lax.fori_loop(0, NCHUNK // _NBUF, body, 0)
    for b in range(_NBUF):
        gd[b].wait()
    plsc.subcore_barrier()

    @pl.when(c == 0)
    def _():
        pltpu.sync_copy(acc.at[sl], out0_hbm.at[sl])

    @pl.when(c != 0)
    def _():
        pltpu.sync_copy(acc.at[sl], out1_hbm.at[sl])


_BLK = 2048


def _dis64(dp_ref):
    dp = dp_ref[...]
    dis_row = lax.rsqrt(dp[0:1, :] + dp[1:2, :] + 1.0)
    ones = jnp.ones((1, DH), jnp.float32)
    return lax.dot_general(dis_row, ones, (((0,), (0,)), ((), ())),
                           preferred_element_type=jnp.float32)


def _tc_a_body(x_ref, w_ref, dp_ref, z0_ref, z1_ref):
    dis = _dis64(dp_ref)
    z = jnp.dot(x_ref[...].astype(jnp.bfloat16),
                w_ref[...].astype(jnp.bfloat16),
                preferred_element_type=jnp.float32)
    z0_ref[...] = z[:, :DH] * dis
    z1_ref[...] = z[:, DH:] * dis


def _tc_b_body(a0_ref, a1_ref, dp_ref, b_ref, w_ref, z0_ref, z1_ref):
    dis = _dis64(dp_ref)
    b = b_ref[...]
    h_lo = jnp.maximum(a0_ref[...] * dis + b[:, :DH], 0.0)
    h_hi = jnp.maximum(a1_ref[...] * dis + b[:, DH:], 0.0)
    h = jnp.concatenate([h_lo, h_hi], axis=1)
    z = jnp.dot(h.astype(jnp.bfloat16), w_ref[...].astype(jnp.bfloat16),
                preferred_element_type=jnp.float32)
    z0_ref[...] = z[:, :DH] * dis
    z1_ref[...] = z[:, DH:] * dis


def _tc_c_body(a0_ref, a1_ref, dp_ref, b_ref, o_ref):
    dis = _dis64(dp_ref)
    b = b_ref[...]
    o_ref[...] = jnp.concatenate(
        [a0_ref[...] * dis + b[:, :DH], a1_ref[...] * dis + b[:, DH:]],
        axis=1)


def _row_spec(blk, d=D):
    return pl.BlockSpec((blk, d), lambda i: (i, 0))


def _full_spec():
    return pl.BlockSpec((D, D), lambda i: (0, 0))


def _deg_spec(blk):
    return pl.BlockSpec((NC, blk), lambda i: (0, i))


def _bias_spec():
    return pl.BlockSpec((1, D), lambda i: (0, 0))


def kernel(x, edge_index, W1, b1, W2, b2):
    f32 = jnp.float32
    src = edge_index[0].astype(jnp.int32)
    dst = edge_index[1].astype(jnp.int32)
    n_extra = E_PAD - src.shape[0]
    src_p = jnp.concatenate([src, jnp.zeros((n_extra,), jnp.int32)])
    dst_p = jnp.concatenate(
        [dst, jnp.full((n_extra,), N_NODES + 100, jnp.int32)])
    dst_deg = dst_p.reshape(NTILE, NCHUNK_DEG, CHUNK)
    idx_agg = jnp.stack([src_p.reshape(NS, NCHUNK, CHUNK),
                         dst_p.reshape(NS, NCHUNK, CHUNK)], axis=2)

    x_pad = jnp.pad(x.astype(f32), ((0, N_PAD - N_NODES), (0, 0)))

    deg_part = _deg_kernel(dst_deg)

    grid = (N_PAD // _BLK,)
    z1_lo, z1_hi = pl.pallas_call(
        _tc_a_body,
        grid=grid,
        in_specs=[_row_spec(_BLK), _full_spec(), _deg_spec(_BLK)],
        out_specs=[_row_spec(_BLK, DH), _row_spec(_BLK, DH)],
        out_shape=[jax.ShapeDtypeStruct((N_PAD, DH), f32),
                   jax.ShapeDtypeStruct((N_PAD, DH), f32)],
    )(x_pad, W1.astype(f32), deg_part)

    s1 = _agg_kernel(z1_lo, z1_hi, idx_agg)
    z2_lo, z2_hi = pl.pallas_call(
        _tc_b_body,
        grid=grid,
        in_specs=[_row_spec(_BLK, DH), _row_spec(_BLK, DH), _deg_spec(_BLK),
                  _bias_spec(), _full_spec()],
        out_specs=[_row_spec(_BLK, DH), _row_spec(_BLK, DH)],
        out_shape=[jax.ShapeDtypeStruct((N_PAD, DH), f32),
                   jax.ShapeDtypeStruct((N_PAD, DH), f32)],
    )(s1[0], s1[1], deg_part, b1.reshape(1, D).astype(f32), W2.astype(f32))

    s2 = _agg_kernel(z2_lo, z2_hi, idx_agg)

    out = pl.pallas_call(
        _tc_c_body,
        grid=grid,
        in_specs=[_row_spec(_BLK, DH), _row_spec(_BLK, DH), _deg_spec(_BLK),
                  _bias_spec()],
        out_specs=_row_spec(_BLK),
        out_shape=jax.ShapeDtypeStruct((N_PAD, D), f32),
    )(s2[0], s2[1], deg_part, b2.reshape(1, D).astype(f32))
    return out[:N_NODES]


# speedup vs baseline: 1.0136x; 1.0136x over previous
import functools

import jax
import jax.numpy as jnp
from jax import lax
from jax.experimental import pallas as pl
from jax.experimental.pallas import tpu as pltpu
from jax.experimental.pallas import tpu_sc as plsc

N_NODES = 10000
N_PAD = 10240
D = 128
DH = D // 2
NC, NS = 2, 16
NTILE = NC * NS
CHUNK = 64
NCHUNK_DEG = 160
NCHUNK = 320
E_PAD = NS * NCHUNK * CHUNK
ROWS_PER_TILE = N_PAD // NS
_NBUF = 8
_NBUF_HBM = 0

_mesh = plsc.VectorSubcoreMesh(core_axis_name="c", subcore_axis_name="s")


@functools.partial(
    pl.kernel,
    out_type=jax.ShapeDtypeStruct((NC, N_PAD), jnp.float32),
    mesh=_mesh,
    scratch_types=[
        pltpu.VMEM((NCHUNK_DEG, CHUNK), jnp.int32),
        pltpu.VMEM((CHUNK,), jnp.float32),
        pltpu.VMEM((ROWS_PER_TILE,), jnp.float32),
        pltpu.VMEM_SHARED((N_PAD,), jnp.float32),
    ],
)
def _deg_kernel(dst_hbm, out_hbm, dst_v, ones_v, zeros_v, acc):
    c = lax.axis_index("c")
    s = lax.axis_index("s")
    wid = c * NS + s
    for k in range(CHUNK // 16):
        ones_v[pl.ds(k * 16, 16)] = jnp.ones((16,), jnp.float32)
    for k in range(ROWS_PER_TILE // 16):
        zeros_v[pl.ds(k * 16, 16)] = jnp.zeros((16,), jnp.float32)
    pltpu.sync_copy(zeros_v, acc.at[pl.ds(s * ROWS_PER_TILE, ROWS_PER_TILE)])
    pltpu.sync_copy(dst_hbm.at[wid], dst_v)
    plsc.subcore_barrier()

    def body(j, carry):
        pltpu.sync_copy(ones_v, acc.at[dst_v.at[j]], add=True)
        return carry

    lax.fori_loop(0, NCHUNK_DEG, body, 0)
    plsc.subcore_barrier()
    pltpu.sync_copy(acc.at[pl.ds(s * ROWS_PER_TILE, ROWS_PER_TILE)],
                    out_hbm.at[c, pl.ds(s * ROWS_PER_TILE, ROWS_PER_TILE)])


@functools.partial(
    pl.kernel,
    out_type=[jax.ShapeDtypeStruct((N_PAD, DH), jnp.float32),
              jax.ShapeDtypeStruct((N_PAD, DH), jnp.float32)],
    mesh=_mesh,
    scratch_types=[
        [pltpu.VMEM((2, CHUNK), jnp.int32)] * _NBUF,
        [pltpu.VMEM((CHUNK, DH), jnp.float32)] * _NBUF,
        pltpu.VMEM_SHARED((N_PAD, DH), jnp.float32),
        pltpu.VMEM_SHARED((N_PAD, DH), jnp.float32),
        [pltpu.SemaphoreType.DMA] * _NBUF,
        [pltpu.SemaphoreType.DMA] * _NBUF,
        [pltpu.SemaphoreType.DMA] * _NBUF,
    ],
    compiler_params=pltpu.CompilerParams(use_tc_tiling_on_sc=False),
)
def _agg_kernel(z0_hbm, z1_hbm, idx_hbm, out0_hbm, out1_hbm, idxb, rows, z_sh,
                acc, isem, gsem, ssem):
    c = lax.axis_index("c")
    s = lax.axis_index("s")
    row0 = s * ROWS_PER_TILE
    sl = pl.ds(row0, ROWS_PER_TILE)

    @pl.when(c == 0)
    def _():
        pltpu.sync_copy(z0_hbm.at[sl], z_sh.at[sl])
        pltpu.sync_copy(z0_hbm.at[sl], acc.at[sl])

    @pl.when(c != 0)
    def _():
        pltpu.sync_copy(z1_hbm.at[sl], z_sh.at[sl])
        pltpu.sync_copy(z1_hbm.at[sl], acc.at[sl])

    def idx_dma(j, b):
        return pltpu.async_copy(idx_hbm.at[s, j], idxb[b], isem[b])

    def gather(b):
        if b < _NBUF_HBM:
            @pl.when(c == 0)
            def _():
                pltpu.async_copy(z0_hbm.at[idxb[b].at[0]], rows[b], gsem[b])

            @pl.when(c != 0)
            def _():
                pltpu.async_copy(z1_hbm.at[idxb[b].at[0]], rows[b], gsem[b])

            return pltpu.make_async_copy(z0_hbm.at[idxb[b].at[0]], rows[b],
                                         gsem[b])
        return pltpu.async_copy(z_sh.at[idxb[b].at[0]], rows[b], gsem[b])

    for b in range(_NBUF):
        idx_dma(b, b)
    plsc.subcore_barrier()
    gd = []
    for b in range(_NBUF):
        pltpu.make_async_copy(idx_hbm.at[s, 0], idxb[b], isem[b]).wait()
        gd.append(gather(b))

    def body(i, carry):
        j = _NBUF * i
        sd = []
        for b in range(_NBUF):
            gd[b].wait()
            sd.append(pltpu.async_copy(rows[b], acc.at[idxb[b].at[1]],
                                       ssem[b], add=True))
        for b in range(_NBUF):
            sd[b].wait()
            idx_dma(jnp.minimum(j + _NBUF + b, NCHUNK - _NBUF + b), b)
        for b in range(_NBUF):
            pltpu.make_async_copy(idx_hbm.at[s, 0], idxb[b], isem[b]).wait()
            gather(b)
        return carry

    lax.fori_loop(0, NCHUNK // _NBUF, body, 0)
    for b in range(_NBUF):
        gd[b].wait()
    plsc.subcore_barrier()

    @pl.when(c == 0)
    def _():
        pltpu.sync_copy(acc.at[sl], out0_hbm.at[sl])

    @pl.when(c != 0)
    def _():
        pltpu.sync_copy(acc.at[sl], out1_hbm.at[sl])


_BLK = 2048


def _dis64(dp_ref):
    dp = dp_ref[...]
    dis_row = lax.rsqrt(dp[0:1, :] + dp[1:2, :] + 1.0)
    ones = jnp.ones((1, DH), jnp.float32)
    return lax.dot_general(dis_row, ones, (((0,), (0,)), ((), ())),
                           preferred_element_type=jnp.float32)


def _tc_a_body(x_ref, w_ref, dp_ref, z0_ref, z1_ref):
    dis = _dis64(dp_ref)
    z = jnp.dot(x_ref[...].astype(jnp.bfloat16),
                w_ref[...].astype(jnp.bfloat16),
                preferred_element_type=jnp.float32)
    z0_ref[...] = z[:, :DH] * dis
    z1_ref[...] = z[:, DH:] * dis


def _tc_b_body(a0_ref, a1_ref, dp_ref, b_ref, w_ref, z0_ref, z1_ref):
    dis = _dis64(dp_ref)
    b = b_ref[...]
    h_lo = jnp.maximum(a0_ref[...] * dis + b[:, :DH], 0.0)
    h_hi = jnp.maximum(a1_ref[...] * dis + b[:, DH:], 0.0)
    h = jnp.concatenate([h_lo, h_hi], axis=1)
    z = jnp.dot(h.astype(jnp.bfloat16), w_ref[...].astype(jnp.bfloat16),
                preferred_element_type=jnp.float32)
    z0_ref[...] = z[:, :DH] * dis
    z1_ref[...] = z[:, DH:] * dis


def _tc_c_body(a0_ref, a1_ref, dp_ref, b_ref, o_ref):
    dis = _dis64(dp_ref)
    b = b_ref[...]
    o_ref[...] = jnp.concatenate(
        [a0_ref[...] * dis + b[:, :DH], a1_ref[...] * dis + b[:, DH:]],
        axis=1)


def _row_spec(blk, d=D):
    return pl.BlockSpec((blk, d), lambda i: (i, 0))


def _full_spec():
    return pl.BlockSpec((D, D), lambda i: (0, 0))


def _deg_spec(blk):
    return pl.BlockSpec((NC, blk), lambda i: (0, i))


def _bias_spec():
    return pl.BlockSpec((1, D), lambda i: (0, 0))


def kernel(x, edge_index, W1, b1, W2, b2):
    f32 = jnp.float32
    src = edge_index[0].astype(jnp.int32)
    dst = edge_index[1].astype(jnp.int32)
    n_extra = E_PAD - src.shape[0]
    src_p = jnp.concatenate([src, jnp.zeros((n_extra,), jnp.int32)])
    dst_p = jnp.concatenate(
        [dst, jnp.full((n_extra,), N_NODES + 100, jnp.int32)])
    dst_deg = dst_p.reshape(NTILE, NCHUNK_DEG, CHUNK)
    idx_agg = jnp.stack([src_p.reshape(NS, NCHUNK, CHUNK),
                         dst_p.reshape(NS, NCHUNK, CHUNK)], axis=2)

    x_pad = jnp.pad(x.astype(f32), ((0, N_PAD - N_NODES), (0, 0)))

    deg_part = _deg_kernel(dst_deg)

    grid = (N_PAD // _BLK,)
    z1_lo, z1_hi = pl.pallas_call(
        _tc_a_body,
        grid=grid,
        in_specs=[_row_spec(_BLK), _full_spec(), _deg_spec(_BLK)],
        out_specs=[_row_spec(_BLK, DH), _row_spec(_BLK, DH)],
        out_shape=[jax.ShapeDtypeStruct((N_PAD, DH), f32),
                   jax.ShapeDtypeStruct((N_PAD, DH), f32)],
    )(x_pad, W1.astype(f32), deg_part)

    s1 = _agg_kernel(z1_lo, z1_hi, idx_agg)
    z2_lo, z2_hi = pl.pallas_call(
        _tc_b_body,
        grid=grid,
        in_specs=[_row_spec(_BLK, DH), _row_spec(_BLK, DH), _deg_spec(_BLK),
                  _bias_spec(), _full_spec()],
        out_specs=[_row_spec(_BLK, DH), _row_spec(_BLK, DH)],
        out_shape=[jax.ShapeDtypeStruct((N_PAD, DH), f32),
                   jax.ShapeDtypeStruct((N_PAD, DH), f32)],
    )(s1[0], s1[1], deg_part, b1.reshape(1, D).astype(f32), W2.astype(f32))

    s2 = _agg_kernel(z2_lo, z2_hi, idx_agg)

    out = pl.pallas_call(
        _tc_c_body,
        grid=grid,
        in_specs=[_row_spec(_BLK, DH), _row_spec(_BLK, DH), _deg_spec(_BLK),
                  _bias_spec()],
        out_specs=_row_spec(_BLK),
        out_shape=jax.ShapeDtypeStruct((N_PAD, D), f32),
    )(s2[0], s2[1], deg_part, b2.reshape(1, D).astype(f32))
    return out[:N_NODES]


# speedup vs baseline: 1.0168x; 1.0031x over previous
import functools

import jax
import jax.numpy as jnp
from jax import lax
from jax.experimental import pallas as pl
from jax.experimental.pallas import tpu as pltpu
from jax.experimental.pallas import tpu_sc as plsc

N_NODES = 10000
N_PAD = 10240
D = 128
DH = D // 2
NC, NS = 2, 16
NTILE = NC * NS
CHUNK = 64
NCHUNK_DEG = 160
NCHUNK = 320
E_PAD = NS * NCHUNK * CHUNK
ROWS_PER_TILE = N_PAD // NS
_NBUF = 8

_mesh = plsc.VectorSubcoreMesh(core_axis_name="c", subcore_axis_name="s")


@functools.partial(
    pl.kernel,
    out_type=jax.ShapeDtypeStruct((NC, N_PAD), jnp.float32),
    mesh=_mesh,
    scratch_types=[
        pltpu.VMEM((NCHUNK_DEG, CHUNK), jnp.int32),
        pltpu.VMEM((CHUNK,), jnp.float32),
        pltpu.VMEM((ROWS_PER_TILE,), jnp.float32),
        pltpu.VMEM_SHARED((N_PAD,), jnp.float32),
    ],
)
def _deg_kernel(dst_hbm, out_hbm, dst_v, ones_v, zeros_v, acc):
    c = lax.axis_index("c")
    s = lax.axis_index("s")
    wid = c * NS + s
    for k in range(CHUNK // 16):
        ones_v[pl.ds(k * 16, 16)] = jnp.ones((16,), jnp.float32)
    for k in range(ROWS_PER_TILE // 16):
        zeros_v[pl.ds(k * 16, 16)] = jnp.zeros((16,), jnp.float32)
    pltpu.sync_copy(zeros_v, acc.at[pl.ds(s * ROWS_PER_TILE, ROWS_PER_TILE)])
    pltpu.sync_copy(dst_hbm.at[wid], dst_v)
    plsc.subcore_barrier()

    def body(j, carry):
        pltpu.sync_copy(ones_v, acc.at[dst_v.at[j]], add=True)
        return carry

    lax.fori_loop(0, NCHUNK_DEG, body, 0)
    plsc.subcore_barrier()
    pltpu.sync_copy(acc.at[pl.ds(s * ROWS_PER_TILE, ROWS_PER_TILE)],
                    out_hbm.at[c, pl.ds(s * ROWS_PER_TILE, ROWS_PER_TILE)])


@functools.partial(
    pl.kernel,
    out_type=[jax.ShapeDtypeStruct((N_PAD, DH), jnp.float32),
              jax.ShapeDtypeStruct((N_PAD, DH), jnp.float32)],
    mesh=_mesh,
    scratch_types=[
        [pltpu.VMEM((2, CHUNK), jnp.int32)] * _NBUF,
        [pltpu.VMEM((CHUNK, DH), jnp.float32)] * _NBUF,
        pltpu.VMEM_SHARED((N_PAD, DH), jnp.float32),
        pltpu.VMEM_SHARED((N_PAD, DH), jnp.float32),
        [pltpu.SemaphoreType.DMA] * _NBUF,
        [pltpu.SemaphoreType.DMA] * _NBUF,
        [pltpu.SemaphoreType.DMA] * _NBUF,
    ],
    compiler_params=pltpu.CompilerParams(use_tc_tiling_on_sc=False),
)
def _agg_kernel(z0_hbm, z1_hbm, idx_hbm, out0_hbm, out1_hbm, idxb, rows, z_sh,
                acc, isem, gsem, ssem):
    c = lax.axis_index("c")
    s = lax.axis_index("s")
    row0 = s * ROWS_PER_TILE
    sl = pl.ds(row0, ROWS_PER_TILE)

    @pl.when(c == 0)
    def _():
        pltpu.sync_copy(z0_hbm.at[sl], z_sh.at[sl])
        pltpu.sync_copy(z0_hbm.at[sl], acc.at[sl])

    @pl.when(c != 0)
    def _():
        pltpu.sync_copy(z1_hbm.at[sl], z_sh.at[sl])
        pltpu.sync_copy(z1_hbm.at[sl], acc.at[sl])

    def idx_dma(j, b):
        return pltpu.async_copy(idx_hbm.at[s, j], idxb[b], isem[b])

    def gather(b):
        return pltpu.async_copy(z_sh.at[idxb[b].at[0]], rows[b], gsem[b])

    for b in range(_NBUF):
        idx_dma(b, b)
    plsc.subcore_barrier()
    gd = []
    for b in range(_NBUF):
        pltpu.make_async_copy(idx_hbm.at[s, 0], idxb[b], isem[b]).wait()
        gd.append(gather(b))

    def body(i, carry):
        j = _NBUF * i
        sd = []
        for b in range(_NBUF):
            gd[b].wait()
            sd.append(pltpu.async_copy(rows[b], acc.at[idxb[b].at[1]],
                                       ssem[b], add=True))
        for b in range(_NBUF):
            sd[b].wait()
            idx_dma(jnp.minimum(j + _NBUF + b, NCHUNK - _NBUF + b), b)
        for b in range(_NBUF):
            pltpu.make_async_copy(idx_hbm.at[s, 0], idxb[b], isem[b]).wait()
            gather(b)
        return carry

    lax.fori_loop(0, NCHUNK // _NBUF, body, 0)
    for b in range(_NBUF):
        gd[b].wait()
    plsc.subcore_barrier()

    @pl.when(c == 0)
    def _():
        pltpu.sync_copy(acc.at[sl], out0_hbm.at[sl])

    @pl.when(c != 0)
    def _():
        pltpu.sync_copy(acc.at[sl], out1_hbm.at[sl])


_BLK = 2048


def _dis64(dp_ref):
    dp = dp_ref[...]
    dis_row = lax.rsqrt(dp[0:1, :] + dp[1:2, :] + 1.0)
    ones = jnp.ones((1, DH), jnp.float32)
    return lax.dot_general(dis_row, ones, (((0,), (0,)), ((), ())),
                           preferred_element_type=jnp.float32)


def _tc_a_body(x_ref, w_ref, dp_ref, z0_ref, z1_ref):
    dis = _dis64(dp_ref)
    z = jnp.dot(x_ref[...].astype(jnp.bfloat16),
                w_ref[...].astype(jnp.bfloat16),
                preferred_element_type=jnp.float32)
    z0_ref[...] = z[:, :DH] * dis
    z1_ref[...] = z[:, DH:] * dis


def _tc_b_body(a0_ref, a1_ref, dp_ref, b_ref, w_ref, z0_ref, z1_ref):
    dis = _dis64(dp_ref)
    b = b_ref[...]
    h_lo = jnp.maximum(a0_ref[...] * dis + b[:, :DH], 0.0)
    h_hi = jnp.maximum(a1_ref[...] * dis + b[:, DH:], 0.0)
    h = jnp.concatenate([h_lo, h_hi], axis=1)
    z = jnp.dot(h.astype(jnp.bfloat16), w_ref[...].astype(jnp.bfloat16),
                preferred_element_type=jnp.float32)
    z0_ref[...] = z[:, :DH] * dis
    z1_ref[...] = z[:, DH:] * dis


def _tc_c_body(a0_ref, a1_ref, dp_ref, b_ref, o_ref):
    dis = _dis64(dp_ref)
    b = b_ref[...]
    o_ref[...] = jnp.concatenate(
        [a0_ref[...] * dis + b[:, :DH], a1_ref[...] * dis + b[:, DH:]],
        axis=1)


def _row_spec(blk, d=D):
    return pl.BlockSpec((blk, d), lambda i: (i, 0))


def _full_spec():
    return pl.BlockSpec((D, D), lambda i: (0, 0))


def _deg_spec(blk):
    return pl.BlockSpec((NC, blk), lambda i: (0, i))


def _bias_spec():
    return pl.BlockSpec((1, D), lambda i: (0, 0))


def kernel(x, edge_index, W1, b1, W2, b2):
    f32 = jnp.float32
    src = edge_index[0].astype(jnp.int32)
    dst = edge_index[1].astype(jnp.int32)
    n_extra = E_PAD - src.shape[0]
    src_p = jnp.concatenate([src, jnp.zeros((n_extra,), jnp.int32)])
    dst_p = jnp.concatenate(
        [dst, jnp.full((n_extra,), N_NODES + 100, jnp.int32)])
    dst_deg = dst_p.reshape(NTILE, NCHUNK_DEG, CHUNK)
    idx_agg = jnp.stack([src_p.reshape(NS, NCHUNK, CHUNK),
                         dst_p.reshape(NS, NCHUNK, CHUNK)], axis=2)

    x_pad = jnp.pad(x.astype(f32), ((0, N_PAD - N_NODES), (0, 0)))

    deg_part = _deg_kernel(dst_deg)

    grid = (N_PAD // _BLK,)
    z1_lo, z1_hi = pl.pallas_call(
        _tc_a_body,
        grid=grid,
        in_specs=[_row_spec(_BLK), _full_spec(), _deg_spec(_BLK)],
        out_specs=[_row_spec(_BLK, DH), _row_spec(_BLK, DH)],
        out_shape=[jax.ShapeDtypeStruct((N_PAD, DH), f32),
                   jax.ShapeDtypeStruct((N_PAD, DH), f32)],
    )(x_pad, W1.astype(f32), deg_part)

    s1 = _agg_kernel(z1_lo, z1_hi, idx_agg)
    z2_lo, z2_hi = pl.pallas_call(
        _tc_b_body,
        grid=grid,
        in_specs=[_row_spec(_BLK, DH), _row_spec(_BLK, DH), _deg_spec(_BLK),
                  _bias_spec(), _full_spec()],
        out_specs=[_row_spec(_BLK, DH), _row_spec(_BLK, DH)],
        out_shape=[jax.ShapeDtypeStruct((N_PAD, DH), f32),
                   jax.ShapeDtypeStruct((N_PAD, DH), f32)],
    )(s1[0], s1[1], deg_part, b1.reshape(1, D).astype(f32), W2.astype(f32))

    s2 = _agg_kernel(z2_lo, z2_hi, idx_agg)

    out = pl.pallas_call(
        _tc_c_body,
        grid=grid,
        in_specs=[_row_spec(_BLK, DH), _row_spec(_BLK, DH), _deg_spec(_BLK),
                  _bias_spec()],
        out_specs=_row_spec(_BLK),
        out_shape=jax.ShapeDtypeStruct((N_PAD, D), f32),
    )(s2[0], s2[1], deg_part, b2.reshape(1, D).astype(f32))
    return out[:N_NODES]
